# baseline (speedup 1.0000x reference)
import jax
import jax.numpy as jnp
from jax.experimental import pallas as pl

_B, _C, _H, _W = 2, 96, 512, 512
_CP, _FP = 16, 8
_NHC, _NWC = _H // _CP, _W // _CP
_LC = _NHC * _NWC
_K = max(1, int(0.25 * _LC))
_BH = 32


def _score_kernel(x_ref, s_ref):
    xb = x_ref[0]
    t = jnp.sum(jnp.abs(xb), axis=(0, 1))[None, :]
    w_ids = jax.lax.broadcasted_iota(jnp.int32, (_W, _NWC), 0) // _CP
    c_ids = jax.lax.broadcasted_iota(jnp.int32, (_W, _NWC), 1)
    pool = (w_ids == c_ids).astype(jnp.float32)
    s_ref[0] = jnp.dot(t, pool, preferred_element_type=jnp.float32,
                       precision=jax.lax.Precision.HIGHEST)


def _mask_kernel(s_ref, m_ref):
    s = s_ref[0, 0][None, :]
    col = jnp.broadcast_to(s, (_LC, _LC))
    row = jnp.transpose(col)
    i_ids = jax.lax.broadcasted_iota(jnp.int32, (_LC, _LC), 0)
    j_ids = jax.lax.broadcasted_iota(jnp.int32, (_LC, _LC), 1)
    beats = (col > row) | ((col == row) & (j_ids < i_ids))
    rank = jnp.sum(beats.astype(jnp.int32), axis=1)[None, :]
    m_ref[0] = (rank < _K).astype(jnp.float32)


def _conv_kernel(x_ref, m_ref, w_ref, b_ref, o_ref):
    xb = x_ref[0]
    n = _BH * _W
    xflat = xb.reshape(_C, n)

    row8 = jax.lax.broadcasted_iota(jnp.int32, (_BH, _W), 0) % _FP
    col8 = jax.lax.broadcasted_iota(jnp.int32, (_BH, _W), 1) % _FP

    acc = jnp.zeros((_C, n), jnp.float32)
    for ky in range(3):
        dy = ky - 1
        for kx in range(3):
            dx = kx - 1
            sh = xb
            if dy != 0:
                sh = jnp.roll(sh, shift=-dy, axis=1)
            if dx != 0:
                sh = jnp.roll(sh, shift=-dx, axis=2)
            valid = None
            if dy == -1:
                valid = row8 != 0
            elif dy == 1:
                valid = row8 != _FP - 1
            if dx == -1:
                v = col8 != 0
                valid = v if valid is None else (valid & v)
            elif dx == 1:
                v = col8 != _FP - 1
                valid = v if valid is None else (valid & v)
            if valid is not None:
                sh = sh * valid.astype(jnp.float32)[None, :, :]
            acc = acc + jnp.dot(w_ref[ky * 3 + kx].astype(jnp.bfloat16),
                                sh.reshape(_C, n).astype(jnp.bfloat16),
                                preferred_element_type=jnp.float32)

    en = jnp.maximum(acc + b_ref[:, :1], 0.0)

    m = m_ref[0, :, 0, :]
    h_ids = jax.lax.broadcasted_iota(jnp.int32, (_BH, _BH // _CP), 0) // _CP
    hr_ids = jax.lax.broadcasted_iota(jnp.int32, (_BH, _BH // _CP), 1)
    rexp = (h_ids == hr_ids).astype(jnp.float32)
    w_ids = jax.lax.broadcasted_iota(jnp.int32, (_NWC, _W), 1) // _CP
    c_ids = jax.lax.broadcasted_iota(jnp.int32, (_NWC, _W), 0)
    cexp = (w_ids == c_ids).astype(jnp.float32)
    mfull = jnp.dot(rexp, jnp.dot(m, cexp, preferred_element_type=jnp.float32),
                    preferred_element_type=jnp.float32)

    sel = mfull.reshape(1, n) > 0.5
    o_ref[0] = jnp.where(sel, en, xflat).reshape(_C, _BH, _W)


def kernel(x, conv_w, conv_b):
    scores = pl.pallas_call(
        _score_kernel,
        grid=(_B, _NHC),
        in_specs=[pl.BlockSpec((1, _C, _CP, _W), lambda b, i: (b, 0, i, 0))],
        out_specs=pl.BlockSpec((1, 1, _NWC), lambda b, i: (b * _NHC + i, 0, 0)),
        out_shape=jax.ShapeDtypeStruct((_B * _NHC, 1, _NWC), jnp.float32),
    )(x)

    mask = pl.pallas_call(
        _mask_kernel,
        grid=(_B,),
        in_specs=[pl.BlockSpec((1, 1, _LC), lambda b: (b, 0, 0))],
        out_specs=pl.BlockSpec((1, 1, _LC), lambda b: (b, 0, 0)),
        out_shape=jax.ShapeDtypeStruct((_B, 1, _LC), jnp.float32),
    )(scores.reshape(_B, 1, _LC))

    mask4 = mask.reshape(_B, _NHC, 1, _NWC)
    wmats = conv_w.transpose(2, 3, 0, 1).reshape(9, _C, _C)
    bcol = conv_b.reshape(_C, 1)

    out = pl.pallas_call(
        _conv_kernel,
        grid=(_B, _H // _BH),
        in_specs=[
            pl.BlockSpec((1, _C, _BH, _W), lambda b, i: (b, 0, i, 0)),
            pl.BlockSpec((1, _BH // _CP, 1, _NWC), lambda b, i: (b, i, 0, 0)),
            pl.BlockSpec((9, _C, _C), lambda b, i: (0, 0, 0)),
            pl.BlockSpec((_C, 1), lambda b, i: (0, 0)),
        ],
        out_specs=pl.BlockSpec((1, _C, _BH, _W), lambda b, i: (b, 0, i, 0)),
        out_shape=jax.ShapeDtypeStruct((_B, _C, _H, _W), jnp.float32),
    )(x, mask4, wmats, bcol)
    return out


# speedup vs baseline: 1.1470x; 1.1470x over previous
import jax
import jax.numpy as jnp
from jax.experimental import pallas as pl

_B, _C, _H, _W = 2, 96, 512, 512
_CP, _FP = 16, 8
_NHC, _NWC = _H // _CP, _W // _CP
_LC = _NHC * _NWC
_K = max(1, int(0.25 * _LC))
_BH = 32


def _score_kernel(x_ref, s_ref):
    xb = x_ref[0]
    t = jnp.sum(jnp.abs(xb), axis=(0, 1))[None, :]
    w_ids = jax.lax.broadcasted_iota(jnp.int32, (_W, _NWC), 0) // _CP
    c_ids = jax.lax.broadcasted_iota(jnp.int32, (_W, _NWC), 1)
    pool = (w_ids == c_ids).astype(jnp.float32)
    s_ref[0] = jnp.dot(t, pool, preferred_element_type=jnp.float32,
                       precision=jax.lax.Precision.HIGHEST)


def _mask_kernel(s_ref, m_ref):
    s = s_ref[0, 0][None, :]
    col = jnp.broadcast_to(s, (_LC, _LC))
    row = jnp.transpose(col)
    i_ids = jax.lax.broadcasted_iota(jnp.int32, (_LC, _LC), 0)
    j_ids = jax.lax.broadcasted_iota(jnp.int32, (_LC, _LC), 1)
    beats = (col > row) | ((col == row) & (j_ids < i_ids))
    rank = jnp.sum(beats.astype(jnp.int32), axis=1)[None, :]
    m_ref[0] = (rank < _K).astype(jnp.float32)


def _conv_kernel(x_ref, m_ref, w_ref, b_ref, o_ref):
    n = _BH * _W
    xflat = x_ref[0]

    pos = jax.lax.broadcasted_iota(jnp.int32, (1, n), 1)
    row8 = (pos // _W) % _FP
    col8 = pos % _FP

    acc = jnp.zeros((_C, n), jnp.float32)
    for ky in range(3):
        dy = ky - 1
        for kx in range(3):
            dx = kx - 1
            shift = -(dy * _W + dx)
            sh = xflat
            if shift != 0:
                sh = jnp.roll(xflat, shift=shift, axis=1)
            valid = None
            if dy == -1:
                valid = row8 != 0
            elif dy == 1:
                valid = row8 != _FP - 1
            if dx == -1:
                v = col8 != 0
                valid = v if valid is None else (valid & v)
            elif dx == 1:
                v = col8 != _FP - 1
                valid = v if valid is None else (valid & v)
            if valid is not None:
                sh = sh * valid.astype(jnp.float32)
            acc = acc + jnp.dot(w_ref[ky * 3 + kx], sh,
                                preferred_element_type=jnp.float32)

    en = jnp.maximum(acc + b_ref[:, :1], 0.0)

    nP = (_BH // _CP) * _NWC
    m = m_ref[0]
    pid = (pos // (_W * _CP)) * _NWC + (pos % _W) // _CP
    j_ids = jax.lax.broadcasted_iota(jnp.int32, (nP, n), 0)
    expand = (jnp.broadcast_to(pid, (nP, n)) == j_ids).astype(jnp.float32)
    mfull = jnp.dot(m, expand, preferred_element_type=jnp.float32)

    o_ref[0] = jnp.where(mfull > 0.5, en, xflat)


def kernel(x, conv_w, conv_b):
    scores = pl.pallas_call(
        _score_kernel,
        grid=(_B, _NHC),
        in_specs=[pl.BlockSpec((1, _C, _CP, _W), lambda b, i: (b, 0, i, 0))],
        out_specs=pl.BlockSpec((1, 1, _NWC), lambda b, i: (b * _NHC + i, 0, 0)),
        out_shape=jax.ShapeDtypeStruct((_B * _NHC, 1, _NWC), jnp.float32),
    )(x)

    mask = pl.pallas_call(
        _mask_kernel,
        grid=(_B,),
        in_specs=[pl.BlockSpec((1, 1, _LC), lambda b: (b, 0, 0))],
        out_specs=pl.BlockSpec((1, 1, _LC), lambda b: (b, 0, 0)),
        out_shape=jax.ShapeDtypeStruct((_B, 1, _LC), jnp.float32),
    )(scores.reshape(_B, 1, _LC))

    nblk = _H // _BH
    npatch = (_BH // _CP) * _NWC
    mask3 = mask.reshape(_B * nblk, 1, npatch)
    wmats = conv_w.transpose(2, 3, 0, 1).reshape(9, _C, _C)
    bcol = conv_b.reshape(_C, 1)
    xflat = x.reshape(_B, _C, _H * _W)

    out = pl.pallas_call(
        _conv_kernel,
        grid=(_B, nblk),
        in_specs=[
            pl.BlockSpec((1, _C, _BH * _W), lambda b, i: (b, 0, i)),
            pl.BlockSpec((1, 1, npatch), lambda b, i: (b * nblk + i, 0, 0)),
            pl.BlockSpec((9, _C, _C), lambda b, i: (0, 0, 0)),
            pl.BlockSpec((_C, 1), lambda b, i: (0, 0)),
        ],
        out_specs=pl.BlockSpec((1, _C, _BH * _W), lambda b, i: (b, 0, i)),
        out_shape=jax.ShapeDtypeStruct((_B, _C, _H * _W), jnp.float32),
    )(xflat, mask3, wmats, bcol)
    return out.reshape(_B, _C, _H, _W)


# speedup vs baseline: 1.3296x; 1.1591x over previous
import jax
import jax.numpy as jnp
from jax.experimental import pallas as pl

_B, _C, _H, _W = 2, 96, 512, 512
_CP, _FP = 16, 8
_NHC, _NWC = _H // _CP, _W // _CP
_LC = _NHC * _NWC
_K = max(1, int(0.25 * _LC))
_BH = 16


def _score_kernel(x_ref, s_ref):
    xb = x_ref[0]
    t = jnp.sum(jnp.abs(xb), axis=(0, 1))[None, :]
    w_ids = jax.lax.broadcasted_iota(jnp.int32, (_W, _NWC), 0) // _CP
    c_ids = jax.lax.broadcasted_iota(jnp.int32, (_W, _NWC), 1)
    pool = (w_ids == c_ids).astype(jnp.float32)
    s_ref[0] = jnp.dot(t, pool, preferred_element_type=jnp.float32,
                       precision=jax.lax.Precision.HIGHEST)


def _mask_kernel(s_ref, m_ref):
    s = s_ref[0, 0][None, :]
    col = jnp.broadcast_to(s, (_LC, _LC))
    row = jnp.transpose(col)
    i_ids = jax.lax.broadcasted_iota(jnp.int32, (_LC, _LC), 0)
    j_ids = jax.lax.broadcasted_iota(jnp.int32, (_LC, _LC), 1)
    beats = (col > row) | ((col == row) & (j_ids < i_ids))
    rank = jnp.sum(beats.astype(jnp.int32), axis=1)[None, :]
    m_ref[0] = (rank < _K).astype(jnp.float32)


def _conv_kernel(x_ref, m_ref, w_ref, b_ref, o_ref):
    n = _BH * _W
    xflat = x_ref[0]

    pos = jax.lax.broadcasted_iota(jnp.int32, (1, n), 1)
    row8 = (pos // _W) % _FP
    col8 = pos % _FP

    shs = []
    for ky in range(3):
        dy = ky - 1
        for kx in range(3):
            dx = kx - 1
            shift = -(dy * _W + dx)
            sh = xflat
            if shift != 0:
                sh = jnp.roll(xflat, shift=shift, axis=1)
            valid = None
            if dy == -1:
                valid = row8 != 0
            elif dy == 1:
                valid = row8 != _FP - 1
            if dx == -1:
                v = col8 != 0
                valid = v if valid is None else (valid & v)
            elif dx == 1:
                v = col8 != _FP - 1
                valid = v if valid is None else (valid & v)
            if valid is not None:
                sh = sh * valid.astype(jnp.float32)
            shs.append(sh)
    stacked = jnp.concatenate(shs, axis=0)
    acc = jnp.dot(w_ref[0], stacked, preferred_element_type=jnp.float32)

    en = jnp.maximum(acc + b_ref[:, :1], 0.0)

    nP = (_BH // _CP) * _NWC
    m = m_ref[0]
    pid = (pos // (_W * _CP)) * _NWC + (pos % _W) // _CP
    j_ids = jax.lax.broadcasted_iota(jnp.int32, (nP, n), 0)
    expand = (jnp.broadcast_to(pid, (nP, n)) == j_ids).astype(jnp.float32)
    mfull = jnp.dot(m, expand, preferred_element_type=jnp.float32)

    o_ref[0] = jnp.where(mfull > 0.5, en, xflat)


def kernel(x, conv_w, conv_b):
    scores = pl.pallas_call(
        _score_kernel,
        grid=(_B, _NHC),
        in_specs=[pl.BlockSpec((1, _C, _CP, _W), lambda b, i: (b, 0, i, 0))],
        out_specs=pl.BlockSpec((1, 1, _NWC), lambda b, i: (b * _NHC + i, 0, 0)),
        out_shape=jax.ShapeDtypeStruct((_B * _NHC, 1, _NWC), jnp.float32),
    )(x)

    mask = pl.pallas_call(
        _mask_kernel,
        grid=(_B,),
        in_specs=[pl.BlockSpec((1, 1, _LC), lambda b: (b, 0, 0))],
        out_specs=pl.BlockSpec((1, 1, _LC), lambda b: (b, 0, 0)),
        out_shape=jax.ShapeDtypeStruct((_B, 1, _LC), jnp.float32),
    )(scores.reshape(_B, 1, _LC))

    nblk = _H // _BH
    npatch = (_BH // _CP) * _NWC
    mask3 = mask.reshape(_B * nblk, 1, npatch)
    wmats = conv_w.transpose(0, 2, 3, 1).reshape(1, _C, 9 * _C)
    bcol = conv_b.reshape(_C, 1)
    xflat = x.reshape(_B, _C, _H * _W)

    out = pl.pallas_call(
        _conv_kernel,
        grid=(_B, nblk),
        in_specs=[
            pl.BlockSpec((1, _C, _BH * _W), lambda b, i: (b, 0, i)),
            pl.BlockSpec((1, 1, npatch), lambda b, i: (b * nblk + i, 0, 0)),
            pl.BlockSpec((1, _C, 9 * _C), lambda b, i: (0, 0, 0)),
            pl.BlockSpec((_C, 1), lambda b, i: (0, 0)),
        ],
        out_specs=pl.BlockSpec((1, _C, _BH * _W), lambda b, i: (b, 0, i)),
        out_shape=jax.ShapeDtypeStruct((_B, _C, _H * _W), jnp.float32),
    )(xflat, mask3, wmats, bcol)
    return out.reshape(_B, _C, _H, _W)
